# baseline (speedup 1.0000x reference)
import jax
import jax.numpy as jnp
from jax.experimental import pallas as pl
from jax.experimental.pallas import tpu as pltpu

_K = 2
_H = 16
_T = 2048
_S = 2048
_BT = 1024
_NT = _T // _BT
_TAIL = 8


def _merge_rows(last):
    col = jax.lax.broadcasted_iota(jnp.int32, (_H, _S), 1)
    row = jax.lax.broadcasted_iota(jnp.int32, (_H, 1), 0)
    maxv = jnp.max(last, axis=1, keepdims=True)
    cand = jnp.min(jnp.where(last == maxv, col, _S), axis=1, keepdims=True)
    onehot = col == cand
    hist = jnp.sum(onehot.astype(jnp.int32), axis=0, keepdims=True)
    cph = jnp.sum(jnp.where(onehot, hist, 0), axis=1, keepdims=True)
    maxc = jnp.max(hist)
    mask = cph == maxc
    do_merge = maxc <= _K
    nmax = jnp.sum(mask.astype(jnp.int32))
    pos = jnp.where(
        nmax <= 4, jnp.int32(0),
        jnp.where(nmax <= 8, jnp.int32(4),
                  jnp.where(nmax <= 12, jnp.int32(5), jnp.int32(12))))
    key = jnp.where(mask, cand * _H + row, _S * _H + row)
    big = jnp.int32(2 * _S * _H)

    def body(_, carry):
        cur, _m = carry
        m = jnp.min(cur)
        return jnp.where(cur == m, big, cur), m

    _, mkey = jax.lax.fori_loop(0, pos + 1, body, (key, jnp.int32(0)))
    shead = jnp.sum(jnp.where(key == mkey, row, 0))
    src = jnp.sum(jnp.where(row == shead, last, 0.0), axis=0, keepdims=True)
    return jnp.where(jnp.logical_and(do_merge, mask), src, last)


def _copy_kernel(x_ref, tail_ref, o_ref):
    h = pl.program_id(0)
    tb = pl.program_id(1)

    o_ref[...] = x_ref[...]

    @pl.when(tb == _NT - 1)
    def _():
        nl = _merge_rows(tail_ref[0, :, _TAIL - 1, :])
        hrow = jax.lax.broadcasted_iota(jnp.int32, (_H, 1), 0) == h
        o_ref[0, 0, _BT - 1:_BT, :] = jnp.sum(
            jnp.where(hrow, nl, 0.0), axis=0, keepdims=True)


def kernel(attention_weight):
    out = pl.pallas_call(
        _copy_kernel,
        grid=(_H, _NT),
        in_specs=[
            pl.BlockSpec((1, 1, _BT, _S), lambda h, tb: (0, h, tb, 0)),
            pl.BlockSpec((1, _H, _TAIL, _S),
                         lambda h, tb: (0, 0, _T // _TAIL - 1, 0)),
        ],
        out_specs=pl.BlockSpec((1, 1, _BT, _S), lambda h, tb: (0, h, tb, 0)),
        out_shape=jax.ShapeDtypeStruct((1, _H, _T, _S), jnp.float32),
        compiler_params=pltpu.CompilerParams(
            dimension_semantics=("parallel", "parallel")),
    )(attention_weight, attention_weight)
    return out
